# baseline (speedup 1.0000x reference)
import math

import jax
import jax.numpy as jnp
from jax.experimental import pallas as pl

_EMBED = 768
_HEADS = 12
_DH = _EMBED // _HEADS
_NB = 4
_ROW_TILE = 256
_HPP = 6
_GHALF = 16.0


def _lsh_attn_kernel(x_ref, wq_ref, bq_ref, wv_ref, bv_ref, hyp_ref, out_ref):
    x = x_ref[...]
    wq = wq_ref[0]
    wv = wv_ref[0]
    bq = bq_ref[0]
    bv = bv_ref[0]
    hyp = hyp_ref[...]

    n = x.shape[0]
    qp = jax.lax.dot_general(
        x, wq, (((1,), (1,)), ((), ())), preferred_element_type=jnp.float32
    ) + bq
    vp = jax.lax.dot_general(
        x.astype(jnp.bfloat16), wv.astype(jnp.bfloat16),
        (((1,), (1,)), ((), ())), preferred_element_type=jnp.float32
    ) + bv

    scale = math.log2(math.e) / math.sqrt(_EMBED)
    iota = jax.lax.broadcasted_iota(jnp.int32, (n, _NB), 1)
    ones_col = jnp.ones((n, 1), dtype=jnp.float32)

    for h in range(_HPP):
        cs = slice(h * _DH, (h + 1) * _DH)
        q = qp[:, cs]
        v = vp[:, cs]

        proj = jnp.dot(q, hyp[:_DH, :], preferred_element_type=jnp.float32)
        proj = proj + hyp[_DH:_DH + 1, :]
        bits = (proj >= 0).astype(jnp.int32)
        bucket = bits[:, 0:1] + 2 * bits[:, 1:2]
        onehot = (bucket == iota).astype(jnp.float32)

        ohg = (onehot * _GHALF).astype(jnp.bfloat16)
        qs_ext = jnp.concatenate(
            [(q * scale).astype(jnp.bfloat16), ohg,
             jnp.full((n, 1), _GHALF, dtype=jnp.bfloat16)], axis=1)
        qk_ext = jnp.concatenate(
            [q.astype(jnp.bfloat16), ohg,
             jnp.full((n, 1), -_GHALF, dtype=jnp.bfloat16)], axis=1)
        vext = jnp.concatenate([v, ones_col], axis=1).astype(jnp.bfloat16)

        bsum = jax.lax.dot_general(
            onehot, jnp.concatenate([v, ones_col], axis=1),
            (((0,), (0,)), ((), ())), preferred_element_type=jnp.float32,
        )
        corr_tab = jnp.sum(bsum, axis=0, keepdims=True) - bsum

        for t in range(n // _ROW_TILE):
            sl = slice(t * _ROW_TILE, (t + 1) * _ROW_TILE)
            s = jax.lax.dot_general(
                qs_ext[sl, :], qk_ext, (((1,), (1,)), ((), ())),
                preferred_element_type=jnp.float32,
            )
            p = jnp.exp2(s).astype(jnp.bfloat16)
            av = jnp.dot(p, vext, preferred_element_type=jnp.float32)
            av = av + jnp.dot(
                onehot[sl, :], corr_tab, preferred_element_type=jnp.float32)
            out_ref[sl, cs] = av[:, :_DH] * (1.0 / av[:, _DH:_DH + 1])


def kernel(x, Wq, bq, Wv, bv, hyperplanes):
    b, n, e = x.shape
    npair = _HEADS // _HPP
    wide = _HPP * _DH
    x2 = x[0]
    wq3 = Wq.reshape(npair, wide, e)
    wv3 = Wv.reshape(npair, wide, e)
    bq3 = bq.reshape(npair, 1, wide)
    bv3 = bv.reshape(npair, 1, wide)

    out = pl.pallas_call(
        _lsh_attn_kernel,
        grid=(npair,),
        in_specs=[
            pl.BlockSpec((n, e), lambda i: (0, 0)),
            pl.BlockSpec((1, wide, e), lambda i: (i, 0, 0)),
            pl.BlockSpec((1, 1, wide), lambda i: (i, 0, 0)),
            pl.BlockSpec((1, wide, e), lambda i: (i, 0, 0)),
            pl.BlockSpec((1, 1, wide), lambda i: (i, 0, 0)),
            pl.BlockSpec((_DH + 1, 2), lambda i: (0, 0)),
        ],
        out_specs=pl.BlockSpec((n, wide), lambda i: (0, i)),
        out_shape=jax.ShapeDtypeStruct((n, e), jnp.float32),
    )(x2, wq3, bq3, wv3, bv3, hyperplanes)

    return out.reshape(b, n, e)


# speedup vs baseline: 1.5069x; 1.5069x over previous
import math

import jax
import jax.numpy as jnp
from jax.experimental import pallas as pl

_EMBED = 768
_HEADS = 12
_DH = _EMBED // _HEADS
_NB = 4
_ROW_TILE = 512
_HPP = 4
_GHALF = 16.0


def _lsh_attn_kernel(x_ref, wq_ref, bq_ref, wv_ref, bv_ref, hyp_ref, out_ref):
    x = x_ref[...]
    wq = wq_ref[0]
    wv = wv_ref[0]
    bq = bq_ref[0]
    bv = bv_ref[0]
    hyp = hyp_ref[...]

    n = x.shape[0]
    qp = jax.lax.dot_general(
        x, wq, (((1,), (1,)), ((), ())), preferred_element_type=jnp.float32
    ) + bq
    vp = jax.lax.dot_general(
        x.astype(jnp.bfloat16), wv.astype(jnp.bfloat16),
        (((1,), (1,)), ((), ())), preferred_element_type=jnp.float32
    ) + bv

    scale = math.log2(math.e) / math.sqrt(_EMBED)
    iota = jax.lax.broadcasted_iota(jnp.int32, (n, _NB), 1)
    ones_col = jnp.ones((n, 1), dtype=jnp.float32)

    for h in range(_HPP):
        cs = slice(h * _DH, (h + 1) * _DH)
        q = qp[:, cs]
        v = vp[:, cs]

        proj = jnp.dot(q, hyp[:_DH, :], preferred_element_type=jnp.float32)
        proj = proj + hyp[_DH:_DH + 1, :]
        bits = (proj >= 0).astype(jnp.int32)
        bucket = bits[:, 0:1] + 2 * bits[:, 1:2]
        onehot = (bucket == iota).astype(jnp.float32)

        ohg = (onehot * _GHALF).astype(jnp.bfloat16)
        qs_ext = jnp.concatenate(
            [(q * scale).astype(jnp.bfloat16), ohg,
             jnp.full((n, 1), _GHALF, dtype=jnp.bfloat16)], axis=1)
        qk_ext = jnp.concatenate(
            [q.astype(jnp.bfloat16), ohg,
             jnp.full((n, 1), -_GHALF, dtype=jnp.bfloat16)], axis=1)
        vext = jnp.concatenate([v, ones_col], axis=1).astype(jnp.bfloat16)

        bsum = jax.lax.dot_general(
            onehot, jnp.concatenate([v, ones_col], axis=1),
            (((0,), (0,)), ((), ())), preferred_element_type=jnp.float32,
        )
        corr_tab = jnp.sum(bsum, axis=0, keepdims=True) - bsum

        for t in range(n // _ROW_TILE):
            sl = slice(t * _ROW_TILE, (t + 1) * _ROW_TILE)
            s = jax.lax.dot_general(
                qs_ext[sl, :], qk_ext, (((1,), (1,)), ((), ())),
                preferred_element_type=jnp.float32,
            )
            p = jnp.exp2(s).astype(jnp.bfloat16)
            av = jnp.dot(p, vext, preferred_element_type=jnp.float32)
            av = av + jnp.dot(
                onehot[sl, :], corr_tab, preferred_element_type=jnp.float32)
            out_ref[sl, cs] = av[:, :_DH] * (1.0 / av[:, _DH:_DH + 1])


def kernel(x, Wq, bq, Wv, bv, hyperplanes):
    b, n, e = x.shape
    npair = _HEADS // _HPP
    wide = _HPP * _DH
    x2 = x[0]
    wq3 = Wq.reshape(npair, wide, e)
    wv3 = Wv.reshape(npair, wide, e)
    bq3 = bq.reshape(npair, 1, wide)
    bv3 = bv.reshape(npair, 1, wide)

    out = pl.pallas_call(
        _lsh_attn_kernel,
        grid=(npair,),
        in_specs=[
            pl.BlockSpec((n, e), lambda i: (0, 0)),
            pl.BlockSpec((1, wide, e), lambda i: (i, 0, 0)),
            pl.BlockSpec((1, 1, wide), lambda i: (i, 0, 0)),
            pl.BlockSpec((1, wide, e), lambda i: (i, 0, 0)),
            pl.BlockSpec((1, 1, wide), lambda i: (i, 0, 0)),
            pl.BlockSpec((_DH + 1, 2), lambda i: (0, 0)),
        ],
        out_specs=pl.BlockSpec((n, wide), lambda i: (0, i)),
        out_shape=jax.ShapeDtypeStruct((n, e), jnp.float32),
    )(x2, wq3, bq3, wv3, bv3, hyperplanes)

    return out.reshape(b, n, e)
